# Initial kernel scaffold; baseline (speedup 1.0000x reference)
import jax
import jax.numpy as jnp
from jax.experimental import pallas as pl


def kernel(features, batch_idx, row_idx, col_idx, group_values, occurrences):
    raise NotImplementedError("write your pallas kernel here")


# speedup vs baseline: 2.3982x; 2.3982x over previous
import functools

import jax
import jax.numpy as jnp
from jax import lax
from jax.experimental import pallas as pl
from jax.experimental.pallas import tpu as pltpu
from jax.experimental.pallas import tpu_sc as plsc

_NW = 32
_K = 64
_R = 32
_L = 16


def _extract(vec, j):
    lane = lax.iota(jnp.int32, _L)
    return jnp.max(jnp.where(lane == j, vec, jnp.zeros_like(vec)))


def _make_sc_call(BE, BU, NF, NNZ_PAD):
    OUT_ROWS = BU + 8
    NCH = NF // _L

    def body(feat_h, bc_h, br_h, g_h, occ_h, par_h, out_h,
             occ_v, rows_v, bc_v, br_v, g_v, val_v, sidx_v, stage_v, par_v):
        wid = lax.axis_index("s") * 2 + lax.axis_index("c")

        pltpu.sync_copy(par_h.at[wid], par_v)
        pvec = par_v[...]
        s = pvec[0]
        e = pvec[1]
        r0 = pvec[2]
        r1 = pvec[3]
        base_al = pvec[4]
        nb = pvec[5]

        pltpu.sync_copy(occ_h, occ_v)

        def zero_stage():
            def zr(r, carry):
                for c in range(NCH):
                    stage_v[r, pl.ds(c * _L, _L)] = jnp.zeros((_L,), jnp.float32)
                return carry
            lax.fori_loop(0, _R, zr, 0)

        def flush(row_base):
            for c in range(_R // _L):
                d = row_base + (c * _L) + lax.iota(jnp.int32, _L)
                sidx_v[pl.ds(c * _L, _L)] = jnp.where(
                    d < r1, d, jnp.int32(OUT_ROWS - 1))
            pltpu.sync_copy(stage_v, out_h.at[sidx_v])
            zero_stage()
            return row_base + _R

        zero_stage()

        def window(w, row_base):
            win_lo = pl.multiple_of(base_al + w * _K, 8)
            pltpu.sync_copy(bc_h.at[pl.ds(win_lo, _K)], bc_v)
            pltpu.sync_copy(br_h.at[pl.ds(win_lo, _K)], br_v)
            pltpu.sync_copy(g_h.at[pl.ds(win_lo, _K)], g_v)
            pltpu.sync_copy(feat_h.at[br_v], rows_v)
            for c in range(_K // _L):
                bcc = bc_v[pl.ds(c * _L, _L)]
                occ_b = plsc.load_gather(occ_v, [bcc])
                val_v[pl.ds(c * _L, _L)] = g_v[pl.ds(c * _L, _L)] / occ_b

            jlo = jnp.maximum(s, win_lo) - win_lo
            jhi = jnp.minimum(e, win_lo + _K) - win_lo

            def entry(j, rb):
                jb = jnp.broadcast_to(j, (_L,))
                bcj = plsc.load_gather(bc_v, [jb])[0]
                rb = lax.while_loop(lambda r: bcj >= r + _R, flush, rb)
                r_loc = bcj - rb
                vb = plsc.load_gather(val_v, [jb])
                for c in range(NCH):
                    plsc.addupdate(
                        stage_v.at[r_loc, pl.ds(c * _L, _L)],
                        vb * rows_v[j, pl.ds(c * _L, _L)])
                return rb

            return lax.fori_loop(jlo, jhi, entry, row_base)

        row_base = lax.fori_loop(0, nb, window, r0)
        lax.while_loop(lambda r: r < r1, flush, row_base)

    return pl.kernel(
        body,
        out_type=jax.ShapeDtypeStruct((OUT_ROWS, NF), jnp.float32),
        mesh=plsc.VectorSubcoreMesh(core_axis_name="c", subcore_axis_name="s",
                                    num_cores=2, num_subcores=16),
        compiler_params=pltpu.CompilerParams(needs_layout_passes=False),
        scratch_types=[
            pltpu.VMEM((BU,), jnp.float32),
            pltpu.VMEM((_K, NF), jnp.float32),
            pltpu.VMEM((_K,), jnp.int32),
            pltpu.VMEM((_K,), jnp.int32),
            pltpu.VMEM((_K,), jnp.float32),
            pltpu.VMEM((_K,), jnp.float32),
            pltpu.VMEM((_R,), jnp.int32),
            pltpu.VMEM((_R, NF), jnp.float32),
            pltpu.VMEM((_L,), jnp.int32),
        ],
    )


def kernel(features, batch_idx, row_idx, col_idx, group_values, occurrences):
    B, NF, E = features.shape
    U = occurrences.shape[1]
    NNZ = batch_idx.shape[0]
    BU, BE = B * U, B * E

    bc = batch_idx * U + col_idx
    br = batch_idx * E + row_idx
    key = bc * (2 ** 15) + br
    key_s, g_s = lax.sort((key, group_values), num_keys=1)
    bc_s = key_s // (2 ** 15)
    br_s = key_s - bc_s * (2 ** 15)

    pad = 2 * _K
    bc_p = jnp.concatenate([bc_s, jnp.full((pad,), BU - 1, jnp.int32)])
    br_p = jnp.concatenate([br_s, jnp.zeros((pad,), jnp.int32)])
    g_p = jnp.concatenate([g_s, jnp.zeros((pad,), jnp.float32)])

    base = (jnp.arange(1, _NW) * NNZ) // _NW
    t_in = jnp.searchsorted(bc_s, bc_s[base], side="left").astype(jnp.int32)
    t_start = jnp.concatenate(
        [jnp.zeros((1,), jnp.int32), t_in, jnp.full((1,), NNZ, jnp.int32)])
    row_in = bc_p[t_in]
    row_start = jnp.concatenate(
        [jnp.zeros((1,), jnp.int32), row_in, jnp.full((1,), BU, jnp.int32)])

    s = t_start[:-1]
    e = t_start[1:]
    base_al = s - (s % 8)
    nb = jnp.where(e > s, (e - base_al + _K - 1) // _K, 0)
    params = jnp.zeros((_NW, _L), jnp.int32)
    params = params.at[:, 0].set(s).at[:, 1].set(e)
    params = params.at[:, 2].set(row_start[:-1]).at[:, 3].set(row_start[1:])
    params = params.at[:, 4].set(base_al).at[:, 5].set(nb)

    feat_t = features.transpose(0, 2, 1).reshape(BE, NF)
    occ_flat = occurrences.reshape(BU)

    sc_call = _make_sc_call(BE, BU, NF, NNZ + pad)
    out_t = sc_call(feat_t, bc_p, br_p, g_p, occ_flat, params)

    return out_t[:BU].reshape(B, U, NF).transpose(0, 2, 1)
